# baseline (speedup 1.0000x reference)
import functools

import jax
import jax.numpy as jnp
from jax import lax
from jax.experimental import pallas as pl
from jax.experimental.pallas import tpu as pltpu
from jax.experimental.pallas import tpu_sc as plsc

N_NODES = 10000
N_PAD = 10112
E_EDGES = 160000
E_PAD = 163840
Q_EDGES = 8192
D_IN = 256
D_HID = 256
D_OUT = 48
NC = 2
NS = 16
CHUNK = 128


def _tc_a_body(deg_ref, emb_ref, w1_ref, g_ref, dinv_ref):
    degs = deg_ref[0, :N_NODES, :] + deg_ref[1, :N_NODES, :]
    deg = jnp.sum(degs, axis=1, keepdims=True) + 1.0
    dinvb = jnp.broadcast_to(lax.rsqrt(deg), (N_NODES, 128))
    dinv_ref[...] = dinvb
    h = jnp.dot(emb_ref[...], w1_ref[...], preferred_element_type=jnp.float32)
    g_ref[0, :N_NODES, :] = h[:, :128] * dinvb
    g_ref[1, :N_NODES, :] = h[:, 128:] * dinvb
    zpad = jnp.zeros((N_PAD - N_NODES, 128), jnp.float32)
    g_ref[0, N_NODES:, :] = zpad
    g_ref[1, N_NODES:, :] = zpad


def _tc_a(deg_parts, emb, W1):
    return pl.pallas_call(
        _tc_a_body,
        out_shape=[
            jax.ShapeDtypeStruct((NC, N_PAD, 128), jnp.float32),
            jax.ShapeDtypeStruct((N_NODES, 128), jnp.float32),
        ],
    )(deg_parts, emb, W1)


def _tc_b_body(s1_ref, dinv_ref, b1_ref, w2_ref, g2_ref):
    dinvb = dinv_ref[...]
    s = jnp.concatenate([s1_ref[0, :N_NODES, :] * dinvb,
                         s1_ref[1, :N_NODES, :] * dinvb], axis=1)
    x1 = jnp.maximum(s + b1_ref[...][None, :], 0.0)
    h2 = jnp.dot(x1, w2_ref[...], preferred_element_type=jnp.float32)
    g2_ref[:N_NODES, :] = h2 * dinvb[:, :D_OUT]
    g2_ref[N_NODES:, :] = jnp.zeros((N_PAD - N_NODES, D_OUT), jnp.float32)


def _tc_b(s1, dinvb, b1, W2):
    return pl.pallas_call(
        _tc_b_body,
        out_shape=jax.ShapeDtypeStruct((N_PAD, D_OUT), jnp.float32),
    )(s1, dinvb, b1, W2)


def _tc_c_body(s2_ref, g2_ref, dinv_ref, b2_ref, x_ref):
    s = s2_ref[0, :N_NODES, :] + s2_ref[1, :N_NODES, :] - g2_ref[:N_NODES, :]
    x_ref[...] = s * dinv_ref[...][:, :D_OUT] + b2_ref[...][None, :]


def _tc_c(s2, g2, dinvb, b2):
    return pl.pallas_call(
        _tc_c_body,
        out_shape=jax.ShapeDtypeStruct((N_NODES, D_OUT), jnp.float32),
    )(s2, g2, dinvb, b2)


def _tc_d_body(xuv_ref, hm_ref, p_ref, r_ref, rt_ref, sn2_ref, out_ref):
    xu = xuv_ref[:Q_EDGES, :]
    xv = xuv_ref[Q_EDGES:, :]
    t2 = jnp.dot(xv * xv, hm_ref[...], preferred_element_type=jnp.float32)
    t3 = jnp.dot(xu * xv, hm_ref[...], preferred_element_type=jnp.float32)
    m = jnp.dot(xu, p_ref[...], preferred_element_type=jnp.float32)
    w = jnp.dot(t2 - 2.0 * t3, r_ref[...], preferred_element_type=jnp.float32)
    delta = w + 2.0 * m - sn2_ref[...]
    e = jnp.exp(jnp.minimum(delta, 60.0))
    ssum = jnp.dot(e, rt_ref[...], preferred_element_type=jnp.float32)
    p0 = 1.0 / (1.0 + ssum)
    out_ref[...] = jnp.sum(p0, axis=1) * (1.0 / 4.0)


def _tc_d(xuv, hm, p, r, rt, sn2):
    return pl.pallas_call(
        _tc_d_body,
        out_shape=jax.ShapeDtypeStruct((Q_EDGES,), jnp.float32),
    )(xuv, hm, p, r, rt, sn2)


_RPT = N_PAD // NS
_HALF_TC = E_PAD // CHUNK // (NC * NS)
_FULL_TC = E_PAD // CHUNK // NS
_G_TC = (2 * Q_EDGES // CHUNK) // (NC * NS)
_CHUNK1 = 64
_MP1_TC = E_PAD // _CHUNK1 // NS
_SB = 5
_SB_TC = _MP1_TC // _SB


def _sc_mesh():
    return plsc.VectorSubcoreMesh(core_axis_name="c", subcore_axis_name="s")


def _deg_stage(dst_p, zeros16):
    @functools.partial(
        pl.kernel,
        out_type=jax.ShapeDtypeStruct((NC, N_PAD, 16), jnp.float32),
        mesh=_sc_mesh(),
        compiler_params=pltpu.CompilerParams(use_tc_tiling_on_sc=False),
        scratch_types=[
            pltpu.VMEM((_HALF_TC, CHUNK), jnp.int32),
            pltpu.VMEM((CHUNK, 16), jnp.float32),
            pltpu.VMEM_SHARED((N_PAD, 16), jnp.float32),
        ],
    )
    def deg_kernel(dst_hbm, zeros_hbm, out_hbm, dst_v, ones_v, acc_sh):
        cid = lax.axis_index("c")
        sid = lax.axis_index("s")
        r0 = sid * _RPT
        pltpu.sync_copy(zeros_hbm.at[pl.ds(r0, _RPT)], acc_sh.at[pl.ds(r0, _RPT)])
        e0 = jnp.where(lax.iota(jnp.int32, 16) == 0, 1.0, 0.0).astype(jnp.float32)

        def fill(i, _):
            ones_v[i, :] = e0
            return 0

        lax.fori_loop(0, CHUNK, fill, 0)
        row0 = (cid * NS + sid) * _HALF_TC
        pltpu.sync_copy(dst_hbm.at[pl.ds(row0, _HALF_TC)], dst_v)
        plsc.subcore_barrier()

        def body(j, _):
            pltpu.sync_copy(ones_v, acc_sh.at[dst_v.at[j]], add=True)
            return 0

        lax.fori_loop(0, _HALF_TC, body, 0)
        plsc.subcore_barrier()
        pltpu.sync_copy(acc_sh.at[pl.ds(r0, _RPT)],
                        out_hbm.at[cid].at[pl.ds(r0, _RPT)])

    return deg_kernel(dst_p, zeros16)


def _mp1_stage(g1, src_p, dst_p):
    @functools.partial(
        pl.kernel,
        out_type=jax.ShapeDtypeStruct((NC, N_PAD, 128), jnp.float32),
        mesh=_sc_mesh(),
        scratch_types=[
            pltpu.VMEM((_SB_TC, _CHUNK1), jnp.int32),
            pltpu.VMEM((_SB_TC, _CHUNK1), jnp.int32),
            pltpu.VMEM((_CHUNK1, 128), jnp.float32),
            pltpu.VMEM((_CHUNK1, 128), jnp.float32),
            pltpu.VMEM_SHARED((N_PAD, 128), jnp.float32),
            pltpu.SemaphoreType.DMA,
            pltpu.SemaphoreType.DMA,
            pltpu.SemaphoreType.DMA,
            pltpu.SemaphoreType.DMA,
        ],
    )
    def mp1_kernel(g_hbm, src_hbm, dst_hbm, out_hbm,
                   src_v, dst_v, rows0, rows1, acc_sh, sem0, sem1, ssem0, ssem1):
        cid = lax.axis_index("c")
        sid = lax.axis_index("s")
        r0 = sid * _RPT
        g_c = g_hbm.at[cid]
        pltpu.sync_copy(g_c.at[pl.ds(r0, _RPT)], acc_sh.at[pl.ds(r0, _RPT)])
        base = sid * _MP1_TC
        plsc.subcore_barrier()

        def sblock(sb, _):
            row0 = base + sb * _SB_TC
            pltpu.sync_copy(src_hbm.at[pl.ds(row0, _SB_TC)], src_v)
            pltpu.sync_copy(dst_hbm.at[pl.ds(row0, _SB_TC)], dst_v)
            pltpu.async_copy(g_c.at[src_v.at[0]], rows0, sem0)

            def body(i, _):
                j0 = 2 * i
                pltpu.make_async_copy(g_c.at[src_v.at[j0]], rows0, sem0).wait()
                pltpu.async_copy(rows0, acc_sh.at[dst_v.at[j0]], ssem0,
                                 add=True)

                @pl.when(i > 0)
                def _():
                    pltpu.make_async_copy(
                        rows1, acc_sh.at[dst_v.at[j0]], ssem1).wait()

                pltpu.async_copy(g_c.at[src_v.at[j0 + 1]], rows1, sem1)
                pltpu.make_async_copy(g_c.at[src_v.at[j0 + 1]], rows1, sem1).wait()
                pltpu.async_copy(rows1, acc_sh.at[dst_v.at[j0 + 1]], ssem1,
                                 add=True)
                pltpu.make_async_copy(
                    rows0, acc_sh.at[dst_v.at[j0]], ssem0).wait()

                @pl.when(i + 1 < _SB_TC // 2)
                def _():
                    pltpu.async_copy(g_c.at[src_v.at[j0 + 2]], rows0, sem0)
                return 0

            lax.fori_loop(0, _SB_TC // 2, body, 0)
            pltpu.make_async_copy(rows1, acc_sh.at[dst_v.at[0]], ssem1).wait()
            return 0

        lax.fori_loop(0, _SB, sblock, 0)
        plsc.subcore_barrier()
        pltpu.sync_copy(acc_sh.at[pl.ds(r0, _RPT)],
                        out_hbm.at[cid].at[pl.ds(r0, _RPT)])

    return mp1_kernel(g1, src_p, dst_p)


def _mp2_stage(g2, src_p, dst_p):
    @functools.partial(
        pl.kernel,
        out_type=jax.ShapeDtypeStruct((NC, N_PAD, D_OUT), jnp.float32),
        mesh=_sc_mesh(),
        compiler_params=pltpu.CompilerParams(use_tc_tiling_on_sc=False),
        scratch_types=[
            pltpu.VMEM((_HALF_TC, CHUNK), jnp.int32),
            pltpu.VMEM((_HALF_TC, CHUNK), jnp.int32),
            pltpu.VMEM((CHUNK, D_OUT), jnp.float32),
            pltpu.VMEM((CHUNK, D_OUT), jnp.float32),
            pltpu.VMEM_SHARED((N_PAD, D_OUT), jnp.float32),
            pltpu.SemaphoreType.DMA,
            pltpu.SemaphoreType.DMA,
            pltpu.SemaphoreType.DMA,
            pltpu.SemaphoreType.DMA,
        ],
    )
    def mp2_kernel(g_hbm, src_hbm, dst_hbm, out_hbm,
                   src_v, dst_v, rows0, rows1, acc_sh, sem0, sem1, ssem0, ssem1):
        cid = lax.axis_index("c")
        sid = lax.axis_index("s")
        r0 = sid * _RPT
        pltpu.sync_copy(g_hbm.at[pl.ds(r0, _RPT)], acc_sh.at[pl.ds(r0, _RPT)])
        row0 = (cid * NS + sid) * _HALF_TC
        pltpu.sync_copy(src_hbm.at[pl.ds(row0, _HALF_TC)], src_v)
        pltpu.sync_copy(dst_hbm.at[pl.ds(row0, _HALF_TC)], dst_v)
        plsc.subcore_barrier()

        pltpu.async_copy(g_hbm.at[src_v.at[0]], rows0, sem0)

        def body(i, _):
            j0 = 2 * i
            pltpu.make_async_copy(g_hbm.at[src_v.at[j0]], rows0, sem0).wait()
            pltpu.async_copy(rows0, acc_sh.at[dst_v.at[j0]], ssem0, add=True)

            @pl.when(i > 0)
            def _():
                pltpu.make_async_copy(
                    rows1, acc_sh.at[dst_v.at[j0]], ssem1).wait()

            pltpu.async_copy(g_hbm.at[src_v.at[j0 + 1]], rows1, sem1)
            pltpu.make_async_copy(g_hbm.at[src_v.at[j0 + 1]], rows1, sem1).wait()
            pltpu.async_copy(rows1, acc_sh.at[dst_v.at[j0 + 1]], ssem1, add=True)
            pltpu.make_async_copy(rows0, acc_sh.at[dst_v.at[j0]], ssem0).wait()

            @pl.when(i + 1 < _HALF_TC // 2)
            def _():
                pltpu.async_copy(g_hbm.at[src_v.at[j0 + 2]], rows0, sem0)
            return 0

        lax.fori_loop(0, _HALF_TC // 2, body, 0)
        pltpu.make_async_copy(rows1, acc_sh.at[dst_v.at[0]], ssem1).wait()
        plsc.subcore_barrier()
        pltpu.sync_copy(acc_sh.at[pl.ds(r0, _RPT)],
                        out_hbm.at[cid].at[pl.ds(r0, _RPT)])

    return mp2_kernel(g2, src_p, dst_p)


def _gather_stage(x, uv):
    @functools.partial(
        pl.kernel,
        out_type=jax.ShapeDtypeStruct((2 * Q_EDGES, D_OUT), jnp.float32),
        mesh=_sc_mesh(),
        compiler_params=pltpu.CompilerParams(use_tc_tiling_on_sc=False),
        scratch_types=[
            pltpu.VMEM((_G_TC, CHUNK), jnp.int32),
            pltpu.VMEM((CHUNK, D_OUT), jnp.float32),
            pltpu.SemaphoreType.DMA,
        ],
    )
    def gather_kernel(x_hbm, uv_hbm, out_hbm, idx_v, rows_v, sem):
        cid = lax.axis_index("c")
        sid = lax.axis_index("s")
        row0 = (cid * NS + sid) * _G_TC
        pltpu.sync_copy(uv_hbm.at[pl.ds(row0, _G_TC)], idx_v)

        def body(j, _):
            pltpu.async_copy(x_hbm.at[idx_v.at[j]], rows_v, sem).wait()
            pltpu.sync_copy(rows_v, out_hbm.at[pl.ds((row0 + j) * CHUNK, CHUNK)])
            return 0

        lax.fori_loop(0, _G_TC, body, 0)

    return gather_kernel(x, uv)


def kernel(edge_index, edges, emb, W1, b1, W2, b2, sentinels):
    src = edge_index[0].astype(jnp.int32)
    dst = edge_index[1].astype(jnp.int32)
    pad_n = E_PAD - E_EDGES
    pad_idx = (jnp.arange(pad_n, dtype=jnp.int32) % 8) + N_NODES
    src_p = jnp.concatenate([src, pad_idx]).reshape(E_PAD // CHUNK, CHUNK)
    dst_p = jnp.concatenate([dst, pad_idx]).reshape(E_PAD // CHUNK, CHUNK)

    deg_parts = _deg_stage(dst_p, jnp.zeros((N_PAD, 16), jnp.float32))
    g1, dinvb = _tc_a(deg_parts, emb, W1)
    s1 = _mp1_stage(g1, src_p.reshape(E_PAD // _CHUNK1, _CHUNK1),
                    dst_p.reshape(E_PAD // _CHUNK1, _CHUNK1))
    g2 = _tc_b(s1, dinvb, b1, W2)
    s2 = _mp2_stage(g2, src_p, dst_p)
    x = _tc_c(s2, g2, dinvb, b2)

    uv = jnp.concatenate([edges[0], edges[1]]).astype(jnp.int32)
    uv = uv.reshape(2 * Q_EDGES // CHUNK, CHUNK)
    xuv = _gather_stage(x, uv)

    n_heads, n_sent, d_head = sentinels.shape
    sent_t = jnp.transpose(sentinels, (2, 0, 1))
    p_mat = (jnp.eye(n_heads, dtype=jnp.float32)[None, :, :, None]
             * sent_t[:, None, :, :]).reshape(D_OUT, n_heads * n_sent)
    sn2 = jnp.sum(sentinels ** 2, axis=-1).reshape(1, n_heads * n_sent)
    hm = (jnp.arange(D_OUT)[:, None] % n_heads
          == jnp.arange(n_heads)[None, :]).astype(jnp.float32)
    r_mat = (jnp.arange(n_heads)[:, None]
             == jnp.arange(n_heads * n_sent)[None, :] // n_sent
             ).astype(jnp.float32)
    return _tc_d(xuv, hm, p_mat, r_mat, r_mat.T, sn2)


# speedup vs baseline: 1.1963x; 1.1963x over previous
import functools

import jax
import jax.numpy as jnp
from jax import lax
from jax.experimental import pallas as pl
from jax.experimental.pallas import tpu as pltpu
from jax.experimental.pallas import tpu_sc as plsc

N_NODES = 10000
N_PAD = 10112
E_EDGES = 160000
E_PAD = 163840
Q_EDGES = 8192
D_IN = 256
D_HID = 256
D_OUT = 48
NC = 2
NS = 16
CHUNK = 128


def _tc_a_body(deg_ref, emb_ref, w1_ref, g_ref, dinv_ref):
    degs = deg_ref[0, :N_NODES, :] + deg_ref[1, :N_NODES, :]
    deg = jnp.sum(degs, axis=1, keepdims=True) + 1.0
    dinvb = jnp.broadcast_to(lax.rsqrt(deg), (N_NODES, 128))
    dinv_ref[...] = dinvb
    h = jnp.dot(emb_ref[...], w1_ref[...], preferred_element_type=jnp.float32)
    g_ref[0, :N_NODES, :] = h[:, :128] * dinvb
    g_ref[1, :N_NODES, :] = h[:, 128:] * dinvb
    zpad = jnp.zeros((N_PAD - N_NODES, 128), jnp.float32)
    g_ref[0, N_NODES:, :] = zpad
    g_ref[1, N_NODES:, :] = zpad


def _tc_a(deg_parts, emb, W1):
    return pl.pallas_call(
        _tc_a_body,
        out_shape=[
            jax.ShapeDtypeStruct((NC, N_PAD, 128), jnp.float32),
            jax.ShapeDtypeStruct((N_NODES, 128), jnp.float32),
        ],
    )(deg_parts, emb, W1)


def _tc_b_body(s1_ref, dinv_ref, b1_ref, w2_ref, g2_ref):
    dinvb = dinv_ref[...]
    s = jnp.concatenate([s1_ref[0, :N_NODES, :] * dinvb,
                         s1_ref[1, :N_NODES, :] * dinvb], axis=1)
    x1 = jnp.maximum(s + b1_ref[...][None, :], 0.0)
    h2 = jnp.dot(x1, w2_ref[...], preferred_element_type=jnp.float32)
    g2_ref[:N_NODES, :] = h2 * dinvb[:, :D_OUT]
    g2_ref[N_NODES:, :] = jnp.zeros((N_PAD - N_NODES, D_OUT), jnp.float32)


def _tc_b(s1, dinvb, b1, W2):
    return pl.pallas_call(
        _tc_b_body,
        out_shape=jax.ShapeDtypeStruct((N_PAD, D_OUT), jnp.float32),
    )(s1, dinvb, b1, W2)


def _tc_c_body(s2_ref, g2_ref, dinv_ref, b2_ref, x_ref):
    s = s2_ref[0, :N_NODES, :] + s2_ref[1, :N_NODES, :] - g2_ref[:N_NODES, :]
    x_ref[...] = s * dinv_ref[...][:, :D_OUT] + b2_ref[...][None, :]


def _tc_c(s2, g2, dinvb, b2):
    return pl.pallas_call(
        _tc_c_body,
        out_shape=jax.ShapeDtypeStruct((N_NODES, D_OUT), jnp.float32),
    )(s2, g2, dinvb, b2)


def _tc_d_body(xuv_ref, hm_ref, p_ref, r_ref, rt_ref, sn2_ref, out_ref):
    xu = xuv_ref[:Q_EDGES, :]
    xv = xuv_ref[Q_EDGES:, :]
    t2 = jnp.dot(xv * xv, hm_ref[...], preferred_element_type=jnp.float32)
    t3 = jnp.dot(xu * xv, hm_ref[...], preferred_element_type=jnp.float32)
    m = jnp.dot(xu, p_ref[...], preferred_element_type=jnp.float32)
    w = jnp.dot(t2 - 2.0 * t3, r_ref[...], preferred_element_type=jnp.float32)
    delta = w + 2.0 * m - sn2_ref[...]
    e = jnp.exp(jnp.minimum(delta, 60.0))
    ssum = jnp.dot(e, rt_ref[...], preferred_element_type=jnp.float32)
    p0 = 1.0 / (1.0 + ssum)
    out_ref[...] = jnp.sum(p0, axis=1) * (1.0 / 4.0)


def _tc_d(xuv, hm, p, r, rt, sn2):
    return pl.pallas_call(
        _tc_d_body,
        out_shape=jax.ShapeDtypeStruct((Q_EDGES,), jnp.float32),
    )(xuv, hm, p, r, rt, sn2)


_RPT = N_PAD // NS
_HALF_TC = E_PAD // CHUNK // (NC * NS)
_FULL_TC = E_PAD // CHUNK // NS
_G_TC = (2 * Q_EDGES // CHUNK) // (NC * NS)
_C1 = 64
_MP1_TC = E_PAD // _C1 // NS
_SB = 5
_SB_TC = _MP1_TC // _SB


def _sc_mesh():
    return plsc.VectorSubcoreMesh(core_axis_name="c", subcore_axis_name="s")


def _ring_gather_scatter(g_c, src_v, dst_v, acc_sh, bufs, gsems, ssems, n_chunks):
    pltpu.async_copy(g_c.at[src_v.at[0]], bufs[0], gsems[0])
    pltpu.async_copy(g_c.at[src_v.at[1]], bufs[1], gsems[1])

    def group(k, _):
        g = k * 4
        for b in range(4):
            i = g + b
            b2 = (b + 2) % 4
            pltpu.make_async_copy(g_c.at[src_v.at[i]], bufs[b], gsems[b]).wait()
            pltpu.async_copy(bufs[b], acc_sh.at[dst_v.at[i]], ssems[b], add=True)
            if b < 2:
                @pl.when(g > 0)
                def _():
                    pltpu.make_async_copy(
                        bufs[b2], acc_sh.at[dst_v.at[i]], ssems[b2]).wait()

                pltpu.async_copy(g_c.at[src_v.at[i + 2]], bufs[b2], gsems[b2])
            else:
                pltpu.make_async_copy(
                    bufs[b2], acc_sh.at[dst_v.at[i]], ssems[b2]).wait()

                @pl.when(g < n_chunks - 4)
                def _():
                    pltpu.async_copy(g_c.at[src_v.at[i + 2]], bufs[b2],
                                     gsems[b2])
        return 0

    lax.fori_loop(0, n_chunks // 4, group, 0)
    pltpu.make_async_copy(bufs[2], acc_sh.at[dst_v.at[0]], ssems[2]).wait()
    pltpu.make_async_copy(bufs[3], acc_sh.at[dst_v.at[0]], ssems[3]).wait()


def _deg_stage(dst_p, zeros16):
    @functools.partial(
        pl.kernel,
        out_type=jax.ShapeDtypeStruct((NC, N_PAD, 16), jnp.float32),
        mesh=_sc_mesh(),
        compiler_params=pltpu.CompilerParams(use_tc_tiling_on_sc=False),
        scratch_types=[
            pltpu.VMEM((_HALF_TC, CHUNK), jnp.int32),
            pltpu.VMEM((CHUNK, 16), jnp.float32),
            pltpu.VMEM_SHARED((N_PAD, 16), jnp.float32),
        ],
    )
    def deg_kernel(dst_hbm, zeros_hbm, out_hbm, dst_v, ones_v, acc_sh):
        cid = lax.axis_index("c")
        sid = lax.axis_index("s")
        r0 = sid * _RPT
        pltpu.sync_copy(zeros_hbm.at[pl.ds(r0, _RPT)], acc_sh.at[pl.ds(r0, _RPT)])
        e0 = jnp.where(lax.iota(jnp.int32, 16) == 0, 1.0, 0.0).astype(jnp.float32)

        def fill(i, _):
            ones_v[i, :] = e0
            return 0

        lax.fori_loop(0, CHUNK, fill, 0)
        row0 = (cid * NS + sid) * _HALF_TC
        pltpu.sync_copy(dst_hbm.at[pl.ds(row0, _HALF_TC)], dst_v)
        plsc.subcore_barrier()

        def body(j, _):
            pltpu.sync_copy(ones_v, acc_sh.at[dst_v.at[j]], add=True)
            return 0

        lax.fori_loop(0, _HALF_TC, body, 0)
        plsc.subcore_barrier()
        pltpu.sync_copy(acc_sh.at[pl.ds(r0, _RPT)],
                        out_hbm.at[cid].at[pl.ds(r0, _RPT)])

    return deg_kernel(dst_p, zeros16)


def _mp1_stage(g1, src_p, dst_p):
    @functools.partial(
        pl.kernel,
        out_type=jax.ShapeDtypeStruct((NC, N_PAD, 128), jnp.float32),
        mesh=_sc_mesh(),
        scratch_types=[
            pltpu.VMEM((_SB_TC, _C1), jnp.int32),
            pltpu.VMEM((_SB_TC, _C1), jnp.int32),
            pltpu.VMEM((_C1, 128), jnp.float32),
            pltpu.VMEM((_C1, 128), jnp.float32),
            pltpu.VMEM((_C1, 128), jnp.float32),
            pltpu.VMEM((_C1, 128), jnp.float32),
            pltpu.VMEM_SHARED((N_PAD, 128), jnp.float32),
            pltpu.SemaphoreType.DMA,
            pltpu.SemaphoreType.DMA,
            pltpu.SemaphoreType.DMA,
            pltpu.SemaphoreType.DMA,
            pltpu.SemaphoreType.DMA,
            pltpu.SemaphoreType.DMA,
            pltpu.SemaphoreType.DMA,
            pltpu.SemaphoreType.DMA,
        ],
    )
    def mp1_kernel(g_hbm, src_hbm, dst_hbm, out_hbm,
                   src_v, dst_v, b0, b1, b2, b3, acc_sh,
                   g0, g1s, g2s, g3, s0, s1, s2, s3):
        cid = lax.axis_index("c")
        sid = lax.axis_index("s")
        r0 = sid * _RPT
        g_c = g_hbm.at[cid]
        pltpu.sync_copy(g_c.at[pl.ds(r0, _RPT)], acc_sh.at[pl.ds(r0, _RPT)])
        base = sid * _MP1_TC
        plsc.subcore_barrier()

        def sblock(sb, _):
            row0 = base + sb * _SB_TC
            pltpu.sync_copy(src_hbm.at[pl.ds(row0, _SB_TC)], src_v)
            pltpu.sync_copy(dst_hbm.at[pl.ds(row0, _SB_TC)], dst_v)
            _ring_gather_scatter(g_c, src_v, dst_v, acc_sh,
                                 [b0, b1, b2, b3], [g0, g1s, g2s, g3],
                                 [s0, s1, s2, s3], _SB_TC)
            return 0

        lax.fori_loop(0, _SB, sblock, 0)
        plsc.subcore_barrier()
        pltpu.sync_copy(acc_sh.at[pl.ds(r0, _RPT)],
                        out_hbm.at[cid].at[pl.ds(r0, _RPT)])

    return mp1_kernel(g1, src_p, dst_p)


def _mp2_stage(g2, src_p, dst_p):
    @functools.partial(
        pl.kernel,
        out_type=jax.ShapeDtypeStruct((NC, N_PAD, D_OUT), jnp.float32),
        mesh=_sc_mesh(),
        compiler_params=pltpu.CompilerParams(use_tc_tiling_on_sc=False),
        scratch_types=[
            pltpu.VMEM((_HALF_TC, CHUNK), jnp.int32),
            pltpu.VMEM((_HALF_TC, CHUNK), jnp.int32),
            pltpu.VMEM((CHUNK, D_OUT), jnp.float32),
            pltpu.VMEM((CHUNK, D_OUT), jnp.float32),
            pltpu.VMEM((CHUNK, D_OUT), jnp.float32),
            pltpu.VMEM((CHUNK, D_OUT), jnp.float32),
            pltpu.VMEM_SHARED((N_PAD, D_OUT), jnp.float32),
            pltpu.SemaphoreType.DMA,
            pltpu.SemaphoreType.DMA,
            pltpu.SemaphoreType.DMA,
            pltpu.SemaphoreType.DMA,
            pltpu.SemaphoreType.DMA,
            pltpu.SemaphoreType.DMA,
            pltpu.SemaphoreType.DMA,
            pltpu.SemaphoreType.DMA,
        ],
    )
    def mp2_kernel(g_hbm, src_hbm, dst_hbm, out_hbm,
                   src_v, dst_v, b0, b1, b2, b3, acc_sh,
                   g0, g1s, g2s, g3, s0, s1, s2, s3):
        cid = lax.axis_index("c")
        sid = lax.axis_index("s")
        r0 = sid * _RPT
        pltpu.sync_copy(g_hbm.at[pl.ds(r0, _RPT)], acc_sh.at[pl.ds(r0, _RPT)])
        row0 = (cid * NS + sid) * _HALF_TC
        pltpu.sync_copy(src_hbm.at[pl.ds(row0, _HALF_TC)], src_v)
        pltpu.sync_copy(dst_hbm.at[pl.ds(row0, _HALF_TC)], dst_v)
        plsc.subcore_barrier()
        _ring_gather_scatter(g_hbm, src_v, dst_v, acc_sh,
                             [b0, b1, b2, b3], [g0, g1s, g2s, g3],
                             [s0, s1, s2, s3], _HALF_TC)
        plsc.subcore_barrier()
        pltpu.sync_copy(acc_sh.at[pl.ds(r0, _RPT)],
                        out_hbm.at[cid].at[pl.ds(r0, _RPT)])

    return mp2_kernel(g2, src_p, dst_p)


def _gather_stage(x, uv):
    @functools.partial(
        pl.kernel,
        out_type=jax.ShapeDtypeStruct((2 * Q_EDGES, D_OUT), jnp.float32),
        mesh=_sc_mesh(),
        compiler_params=pltpu.CompilerParams(use_tc_tiling_on_sc=False),
        scratch_types=[
            pltpu.VMEM((_G_TC, CHUNK), jnp.int32),
            pltpu.VMEM((CHUNK, D_OUT), jnp.float32),
            pltpu.SemaphoreType.DMA,
        ],
    )
    def gather_kernel(x_hbm, uv_hbm, out_hbm, idx_v, rows_v, sem):
        cid = lax.axis_index("c")
        sid = lax.axis_index("s")
        row0 = (cid * NS + sid) * _G_TC
        pltpu.sync_copy(uv_hbm.at[pl.ds(row0, _G_TC)], idx_v)

        def body(j, _):
            pltpu.async_copy(x_hbm.at[idx_v.at[j]], rows_v, sem).wait()
            pltpu.sync_copy(rows_v, out_hbm.at[pl.ds((row0 + j) * CHUNK, CHUNK)])
            return 0

        lax.fori_loop(0, _G_TC, body, 0)

    return gather_kernel(x, uv)


def kernel(edge_index, edges, emb, W1, b1, W2, b2, sentinels):
    src = edge_index[0].astype(jnp.int32)
    dst = edge_index[1].astype(jnp.int32)
    pad_n = E_PAD - E_EDGES
    pad_idx = (jnp.arange(pad_n, dtype=jnp.int32) % 8) + N_NODES
    src_p = jnp.concatenate([src, pad_idx]).reshape(E_PAD // CHUNK, CHUNK)
    dst_p = jnp.concatenate([dst, pad_idx]).reshape(E_PAD // CHUNK, CHUNK)

    deg_parts = _deg_stage(dst_p, jnp.zeros((N_PAD, 16), jnp.float32))
    g1, dinvb = _tc_a(deg_parts, emb, W1)
    s1 = _mp1_stage(g1, src_p.reshape(E_PAD // _C1, _C1),
                    dst_p.reshape(E_PAD // _C1, _C1))
    g2 = _tc_b(s1, dinvb, b1, W2)
    s2 = _mp2_stage(g2, src_p, dst_p)
    x = _tc_c(s2, g2, dinvb, b2)

    uv = jnp.concatenate([edges[0], edges[1]]).astype(jnp.int32)
    uv = uv.reshape(2 * Q_EDGES // CHUNK, CHUNK)
    xuv = _gather_stage(x, uv)

    n_heads, n_sent, d_head = sentinels.shape
    sent_t = jnp.transpose(sentinels, (2, 0, 1))
    p_mat = (jnp.eye(n_heads, dtype=jnp.float32)[None, :, :, None]
             * sent_t[:, None, :, :]).reshape(D_OUT, n_heads * n_sent)
    sn2 = jnp.sum(sentinels ** 2, axis=-1).reshape(1, n_heads * n_sent)
    hm = (jnp.arange(D_OUT)[:, None] % n_heads
          == jnp.arange(n_heads)[None, :]).astype(jnp.float32)
    r_mat = (jnp.arange(n_heads)[:, None]
             == jnp.arange(n_heads * n_sent)[None, :] // n_sent
             ).astype(jnp.float32)
    return _tc_d(xuv, hm, p_mat, r_mat, r_mat.T, sn2)
